# Initial kernel scaffold; baseline (speedup 1.0000x reference)
import functools

import jax
import jax.numpy as jnp
from jax import lax
from jax.experimental import pallas as pl
from jax.experimental.pallas import tpu as pltpu
from jax.experimental.pallas import tpu_sc as plsc

L = 16
NC = 2
NS = 16

N = 10000
E = 160000
D = 256

RPT = 320
HALF = RPT * NS
NPAD = HALF * NC
DUMMY = HALF
ZPT = 328
ACC_ROWS = ZPT * NS

EPT = E // NS
GRP = EPT // L

DCHG = 25
DCH = DCHG * L
DNCH = GRP // DCHG

K = 128
CBUF = EPT + K

_MESH = plsc.VectorSubcoreMesh(core_axis_name="c", subcore_axis_name="s")


def _rsqrt_newton(d):
    i = plsc.bitcast(d, jnp.int32)
    i = 0x5F3759DF - lax.shift_right_logical(i, 1)
    y = plsc.bitcast(i, jnp.float32)
    for _ in range(3):
        y = y * (1.5 - 0.5 * d * y * y)
    return y


def _degree_norm_body(src_hbm, dst_hbm, ns_out, nd_out,
                      acc_o, acc_i, ebuf_s, ebuf_d, idx_s, idx_d,
                      ones_b, zbuf, dbuf, nbuf):
    c = lax.axis_index("c")
    s = lax.axis_index("s")
    off = c * HALF

    def fill_ones(i, _):
        ones_b[i, :] = jnp.full((L,), 1.0, jnp.float32)
        return 0
    lax.fori_loop(0, DCH, fill_ones, 0)

    def fill_zero(i, _):
        zbuf[i, :] = jnp.zeros((L,), jnp.float32)
        return 0
    lax.fori_loop(0, ZPT, fill_zero, 0)

    pltpu.sync_copy(zbuf, acc_o.at[pl.ds(s * ZPT, ZPT)])
    pltpu.sync_copy(zbuf, acc_i.at[pl.ds(s * ZPT, ZPT)])
    plsc.subcore_barrier()

    pltpu.sync_copy(src_hbm.at[pl.ds(s * EPT, EPT)], ebuf_s)
    pltpu.sync_copy(dst_hbm.at[pl.ds(s * EPT, EPT)], ebuf_d)

    def chunk(ci, _):
        def grp(g, _):
            b = ci * DCH + g * L
            sv = ebuf_s[pl.ds(b, L)]
            dv = ebuf_d[pl.ds(b, L)]
            ls = sv - off
            ld = dv - off
            ms = (ls >= 0) & (ls < HALF)
            md = (ld >= 0) & (ld < HALF)
            idx_s[pl.ds(g * L, L)] = jnp.where(ms, ls, DUMMY)
            idx_d[pl.ds(g * L, L)] = jnp.where(md, ld, DUMMY)
            return 0
        lax.fori_loop(0, DCHG, grp, 0)
        pltpu.sync_copy(ones_b, acc_o.at[idx_s], add=True)
        pltpu.sync_copy(ones_b, acc_i.at[idx_d], add=True)
        return 0
    lax.fori_loop(0, DNCH, chunk, 0)
    plsc.subcore_barrier()

    zero16 = jnp.zeros((L,), jnp.int32)
    for acc, out in ((acc_o, ns_out), (acc_i, nd_out)):
        pltpu.sync_copy(acc.at[pl.ds(s * RPT, RPT)], dbuf)

        def norm(k, _):
            rix = lax.iota(jnp.int32, L) + k * L
            deg = plsc.load_gather(dbuf, [rix, zero16])
            nbuf[pl.ds(k * L, L)] = _rsqrt_newton(jnp.maximum(deg, 1.0))
            return 0
        lax.fori_loop(0, RPT // L, norm, 0)
        pltpu.sync_copy(nbuf, out.at[pl.ds(off + s * RPT, RPT)])


@functools.partial(
    pl.kernel,
    out_type=(jax.ShapeDtypeStruct((NPAD,), jnp.float32),
              jax.ShapeDtypeStruct((NPAD,), jnp.float32)),
    mesh=_MESH,
    scratch_types=[
        pltpu.VMEM_SHARED((ACC_ROWS, L), jnp.float32),
        pltpu.VMEM_SHARED((ACC_ROWS, L), jnp.float32),
        pltpu.VMEM((EPT,), jnp.int32),
        pltpu.VMEM((EPT,), jnp.int32),
        pltpu.VMEM((DCH,), jnp.int32),
        pltpu.VMEM((DCH,), jnp.int32),
        pltpu.VMEM((DCH, L), jnp.float32),
        pltpu.VMEM((ZPT, L), jnp.float32),
        pltpu.VMEM((RPT, L), jnp.float32),
        pltpu.VMEM((RPT,), jnp.float32),
    ],
)
def _degree_norms(src_hbm, dst_hbm, ns_out, nd_out, *scratch):
    _degree_norm_body(src_hbm, dst_hbm, ns_out, nd_out, *scratch)


def _agg_body(xs_hbm, src_hbm, dst_hbm, ew_hbm, ns_hbm, out_hbm,
              acc, ns_t, ebuf_s, ebuf_d, ebuf_w, csrc, cdst, cw,
              rows, gsrc, gdst, sem):
    c = lax.axis_index("c")
    s = lax.axis_index("s")
    off = c * HALF

    def zrow(r, _):
        for j in range(D // L):
            rows[r, pl.ds(j * L, L)] = jnp.zeros((L,), jnp.float32)
        return 0
    lax.fori_loop(0, K, zrow, 0)
    pltpu.sync_copy(rows, acc.at[pl.ds(s * ZPT, K)])
    pltpu.sync_copy(rows, acc.at[pl.ds(s * ZPT + K, K)])
    pltpu.sync_copy(rows.at[pl.ds(0, ZPT - 2 * K)],
                    acc.at[pl.ds(s * ZPT + 2 * K, ZPT - 2 * K)])

    pltpu.sync_copy(ns_hbm, ns_t)
    pltpu.sync_copy(src_hbm.at[pl.ds(s * EPT, EPT)], ebuf_s)
    pltpu.sync_copy(dst_hbm.at[pl.ds(s * EPT, EPT)], ebuf_d)
    pltpu.sync_copy(ew_hbm.at[pl.ds(s * EPT, EPT)], ebuf_w)

    dummy16 = jnp.full((L,), DUMMY, jnp.int32)
    zero16i = jnp.zeros((L,), jnp.int32)
    zero16f = jnp.zeros((L,), jnp.float32)

    def pre(g, _):
        cdst[pl.ds(g * L, L)] = dummy16
        csrc[pl.ds(g * L, L)] = zero16i
        cw[pl.ds(g * L, L)] = zero16f
        return 0
    lax.fori_loop(0, CBUF // L, pre, 0)
    plsc.subcore_barrier()

    def scan(g, cnt):
        sv = ebuf_s[pl.ds(g * L, L)]
        dv = ebuf_d[pl.ds(g * L, L)]
        wv = ebuf_w[pl.ds(g * L, L)]
        ld = dv - off
        m = (ld >= 0) & (ld < HALF)
        w = wv * plsc.load_gather(ns_t, [sv])
        plsc.store_compressed(csrc.at[pl.ds(cnt, L)], sv, mask=m)
        plsc.store_compressed(cdst.at[pl.ds(cnt, L)], ld, mask=m)
        plsc.store_compressed(cw.at[pl.ds(cnt, L)], w, mask=m)
        return cnt + jnp.max(plsc.all_reduce_population_count(m))
    cnt = lax.fori_loop(0, GRP, scan, jnp.int32(0))

    nchunks = (cnt + K - 1) // K

    def chunk(ci, _):
        base = ci * K

        def cpy(t, _):
            gsrc[pl.ds(t * L, L)] = csrc[pl.ds(base + t * L, L)]
            gdst[pl.ds(t * L, L)] = cdst[pl.ds(base + t * L, L)]
            return 0
        lax.fori_loop(0, K // L, cpy, 0)

        pltpu.async_copy(xs_hbm.at[gsrc], rows, sem).wait()

        def mul(e, _):
            w = plsc.load_gather(cw, [jnp.full((L,), base, jnp.int32) + e])
            for j in range(D // L):
                rows[e, pl.ds(j * L, L)] = rows[e, pl.ds(j * L, L)] * w
            return 0
        lax.fori_loop(0, K, mul, 0)

        pltpu.sync_copy(rows, acc.at[gdst], add=True)
        return 0
    lax.fori_loop(0, nchunks, chunk, 0)
    plsc.subcore_barrier()

    pltpu.sync_copy(acc.at[pl.ds(s * RPT, RPT)],
                    out_hbm.at[pl.ds(off + s * RPT, RPT)])


@functools.partial(
    pl.kernel,
    out_type=jax.ShapeDtypeStruct((NPAD, D), jnp.float32),
    mesh=_MESH,
    scratch_types=[
        pltpu.VMEM_SHARED((ACC_ROWS, D), jnp.float32),
        pltpu.VMEM((NPAD,), jnp.float32),
        pltpu.VMEM((EPT,), jnp.int32),
        pltpu.VMEM((EPT,), jnp.int32),
        pltpu.VMEM((EPT,), jnp.float32),
        pltpu.VMEM((CBUF,), jnp.int32),
        pltpu.VMEM((CBUF,), jnp.int32),
        pltpu.VMEM((CBUF,), jnp.float32),
        pltpu.VMEM((K, D), jnp.float32),
        pltpu.VMEM((K,), jnp.int32),
        pltpu.VMEM((K,), jnp.int32),
        pltpu.SemaphoreType.DMA,
    ],
)
def _agg(xs_hbm, src_hbm, dst_hbm, ew_hbm, ns_hbm, out_hbm, *scratch):
    _agg_body(xs_hbm, src_hbm, dst_hbm, ew_hbm, ns_hbm, out_hbm, *scratch)


def _dense_kernel(nd_ref, a_ref, w_ref, b_ref, o_ref):
    a = a_ref[...] * nd_ref[...]
    acc = jnp.dot(a, w_ref[...], preferred_element_type=jnp.float32)
    o_ref[...] = jnp.maximum(acc + b_ref[...], 0.0)


def _dense(agg, nd, W, b):
    BN = 1000
    return pl.pallas_call(
        _dense_kernel,
        grid=(N // BN,),
        in_specs=[
            pl.BlockSpec((BN, 1), lambda i: (i, 0)),
            pl.BlockSpec((BN, D), lambda i: (i, 0)),
            pl.BlockSpec((D, D), lambda i: (0, 0)),
            pl.BlockSpec((1, D), lambda i: (0, 0)),
        ],
        out_specs=pl.BlockSpec((BN, D), lambda i: (i, 0)),
        out_shape=jax.ShapeDtypeStruct((N, D), jnp.float32),
    )(nd, agg, W, b)


def kernel(x, edge_index, edge_weight, W1, b1, W2, b2):
    src = edge_index[0]
    dst = edge_index[1]
    ns, nd = _degree_norms(src, dst)
    nd2 = nd[:N].reshape(N, 1)
    b1r = b1.reshape(1, D)
    b2r = b2.reshape(1, D)
    agg1 = _agg(x, src, dst, edge_weight, ns)[:N]
    h1 = _dense(agg1, nd2, W1, b1r)
    agg2 = _agg(h1, src, dst, edge_weight, ns)[:N]
    return _dense(agg2, nd2, W2, b2r)


# speedup vs baseline: 3.3439x; 3.3439x over previous
import functools

import jax
import jax.numpy as jnp
from jax import lax
from jax.experimental import pallas as pl
from jax.experimental.pallas import tpu as pltpu
from jax.experimental.pallas import tpu_sc as plsc

L = 16
NC = 2
NS = 16

N = 10000
E = 160000
D = 256

RPT = 320
HALF = RPT * NS
NPAD = HALF * NC
DUMMY = HALF
ZPT = 321
ACC_ROWS = ZPT * NS

EPT = E // NS
GRP = EPT // L

DCHG = 25
DCH = DCHG * L
DNCH = GRP // DCHG

K = 64
EBUF = EPT + K

_MESH = plsc.VectorSubcoreMesh(core_axis_name="c", subcore_axis_name="s")
_SC_PARAMS = pltpu.CompilerParams(needs_layout_passes=False,
                                  use_tc_tiling_on_sc=False)


def _rsqrt_newton(d):
    i = lax.bitcast_convert_type(d, jnp.int32)
    i = 0x5F3759DF - lax.shift_right_logical(i, 1)
    y = lax.bitcast_convert_type(i, jnp.float32)
    for _ in range(3):
        y = y * (1.5 - 0.5 * d * y * y)
    return y


def _degree_norm_body(src_hbm, dst_hbm, ns_out, nd_out,
                      acc_o, acc_i, ebuf_s, ebuf_d, idx_s, idx_d,
                      ones_b, zbuf, dbuf, nbuf):
    c = lax.axis_index("c")
    s = lax.axis_index("s")
    off = c * HALF

    def fill_ones(i, _):
        ones_b[i, :] = jnp.full((L,), 1.0, jnp.float32)
        return 0
    lax.fori_loop(0, DCH, fill_ones, 0)

    def fill_zero(i, _):
        zbuf[i, :] = jnp.zeros((L,), jnp.float32)
        return 0
    lax.fori_loop(0, ZPT, fill_zero, 0)

    pltpu.sync_copy(zbuf, acc_o.at[pl.ds(s * ZPT, ZPT)])
    pltpu.sync_copy(zbuf, acc_i.at[pl.ds(s * ZPT, ZPT)])
    plsc.subcore_barrier()

    pltpu.sync_copy(src_hbm.at[pl.ds(s * EPT, EPT)], ebuf_s)
    pltpu.sync_copy(dst_hbm.at[pl.ds(s * EPT, EPT)], ebuf_d)

    def chunk(ci, _):
        def grp(g, _):
            b = ci * DCH + g * L
            sv = ebuf_s[pl.ds(b, L)]
            dv = ebuf_d[pl.ds(b, L)]
            ls = sv - off
            ld = dv - off
            ms = (ls >= 0) & (ls < HALF)
            md = (ld >= 0) & (ld < HALF)
            idx_s[pl.ds(g * L, L)] = jnp.where(ms, ls, DUMMY)
            idx_d[pl.ds(g * L, L)] = jnp.where(md, ld, DUMMY)
            return 0
        lax.fori_loop(0, DCHG, grp, 0)
        pltpu.sync_copy(ones_b, acc_o.at[idx_s], add=True)
        pltpu.sync_copy(ones_b, acc_i.at[idx_d], add=True)
        return 0
    lax.fori_loop(0, DNCH, chunk, 0)
    plsc.subcore_barrier()

    m0 = lax.iota(jnp.int32, L) == 0
    for acc, out in ((acc_o, ns_out), (acc_i, nd_out)):
        pltpu.sync_copy(acc.at[pl.ds(s * RPT, RPT)], dbuf)

        def norm(r, _):
            deg = dbuf[r, :]
            y = _rsqrt_newton(jnp.maximum(deg, 1.0))
            plsc.store_scatter(nbuf, [jnp.full((L,), r, jnp.int32)], y,
                               mask=m0)
            return 0
        lax.fori_loop(0, RPT, norm, 0)
        pltpu.sync_copy(nbuf, out.at[pl.ds(off + s * RPT, RPT)])


@functools.partial(
    pl.kernel,
    out_type=(jax.ShapeDtypeStruct((NPAD,), jnp.float32),
              jax.ShapeDtypeStruct((NPAD,), jnp.float32)),
    mesh=_MESH,
    scratch_types=[
        pltpu.VMEM_SHARED((ACC_ROWS, L), jnp.float32),
        pltpu.VMEM_SHARED((ACC_ROWS, L), jnp.float32),
        pltpu.VMEM((EPT,), jnp.int32),
        pltpu.VMEM((EPT,), jnp.int32),
        pltpu.VMEM((DCH,), jnp.int32),
        pltpu.VMEM((DCH,), jnp.int32),
        pltpu.VMEM((DCH, L), jnp.float32),
        pltpu.VMEM((ZPT, L), jnp.float32),
        pltpu.VMEM((RPT, L), jnp.float32),
        pltpu.VMEM((RPT,), jnp.float32),
    ],
    compiler_params=_SC_PARAMS,
)
def _degree_norms(src_hbm, dst_hbm, ns_out, nd_out, *scratch):
    _degree_norm_body(src_hbm, dst_hbm, ns_out, nd_out, *scratch)


def _agg_body(xs_hbm, src_hbm, dst_hbm, ew_hbm, out_hbm,
              acc, ebuf_s, ebuf_d, ebuf_w, rows, gdst, sem):
    c = lax.axis_index("c")
    s = lax.axis_index("s")
    off = c * HALF

    def zrow(r, _):
        for j in range(D // L):
            rows[r, pl.ds(j * L, L)] = jnp.zeros((L,), jnp.float32)
        return 0
    lax.fori_loop(0, K, zrow, 0)
    for i in range(ZPT // K):
        pltpu.sync_copy(rows, acc.at[pl.ds(s * ZPT + i * K, K)])
    if ZPT % K:
        pltpu.sync_copy(rows.at[pl.ds(0, ZPT % K)],
                        acc.at[pl.ds(s * ZPT + (ZPT // K) * K, ZPT % K)])

    pltpu.sync_copy(src_hbm.at[pl.ds(s * EPT, EPT)],
                    ebuf_s.at[pl.ds(0, EPT)])
    pltpu.sync_copy(dst_hbm.at[pl.ds(s * EPT, EPT)],
                    ebuf_d.at[pl.ds(0, EPT)])
    pltpu.sync_copy(ew_hbm.at[pl.ds(s * EPT, EPT)],
                    ebuf_w.at[pl.ds(0, EPT)])
    plsc.subcore_barrier()

    def scan(g, cnt):
        sv = ebuf_s[pl.ds(g * L, L)]
        dv = ebuf_d[pl.ds(g * L, L)]
        wv = ebuf_w[pl.ds(g * L, L)]
        ld = dv - off
        m = (ld >= 0) & (ld < HALF)
        plsc.store_compressed(ebuf_s.at[pl.ds(cnt, L)], sv, mask=m)
        plsc.store_compressed(ebuf_d.at[pl.ds(cnt, L)], ld, mask=m)
        plsc.store_compressed(ebuf_w.at[pl.ds(cnt, L)], wv, mask=m)
        return cnt + jnp.max(plsc.all_reduce_population_count(m))
    cnt = lax.fori_loop(0, GRP, scan, jnp.int32(0))

    dummy16 = jnp.full((L,), DUMMY, jnp.int32)

    def tail(t, _):
        ebuf_d[pl.ds(cnt + t * L, L)] = dummy16
        return 0
    lax.fori_loop(0, K // L, tail, 0)

    nchunks = (cnt + K - 1) // K

    def chunk(ci, _):
        base = ci * K

        def cpy(t, _):
            gdst[pl.ds(t * L, L)] = ebuf_d[pl.ds(base + t * L, L)]
            return 0
        lax.fori_loop(0, K // L, cpy, 0)

        pltpu.async_copy(xs_hbm.at[ebuf_s.at[pl.ds(base, K)]], rows,
                         sem).wait()

        def mul(e, _):
            w = plsc.load_gather(ebuf_w, [jnp.full((L,), base, jnp.int32) + e])
            for j in range(D // L):
                rows[e, pl.ds(j * L, L)] = rows[e, pl.ds(j * L, L)] * w
            return 0
        lax.fori_loop(0, K, mul, 0)

        pltpu.sync_copy(rows, acc.at[gdst], add=True)
        return 0
    lax.fori_loop(0, nchunks, chunk, 0)
    plsc.subcore_barrier()

    pltpu.sync_copy(acc.at[pl.ds(s * RPT, RPT)],
                    out_hbm.at[pl.ds(off + s * RPT, RPT)])


@functools.partial(
    pl.kernel,
    out_type=jax.ShapeDtypeStruct((NPAD, D), jnp.float32),
    mesh=_MESH,
    scratch_types=[
        pltpu.VMEM_SHARED((ACC_ROWS, D), jnp.float32),
        pltpu.VMEM((EBUF,), jnp.int32),
        pltpu.VMEM((EBUF,), jnp.int32),
        pltpu.VMEM((EBUF,), jnp.float32),
        pltpu.VMEM((K, D), jnp.float32),
        pltpu.VMEM((K,), jnp.int32),
        pltpu.SemaphoreType.DMA,
    ],
    compiler_params=_SC_PARAMS,
)
def _agg(xs_hbm, src_hbm, dst_hbm, ew_hbm, out_hbm, *scratch):
    _agg_body(xs_hbm, src_hbm, dst_hbm, ew_hbm, out_hbm, *scratch)


def _dense_kernel(nd_ref, a_ref, w_ref, b_ref, o_ref):
    a = a_ref[...] * nd_ref[...]
    acc = jnp.dot(a, w_ref[...], preferred_element_type=jnp.float32)
    o_ref[...] = jnp.maximum(acc + b_ref[...], 0.0)


def _dense_scaled_kernel(nd_ref, a_ref, w_ref, b_ref, ns_ref, o_ref):
    a = a_ref[...] * nd_ref[...]
    acc = jnp.dot(a, w_ref[...], preferred_element_type=jnp.float32)
    o_ref[...] = jnp.maximum(acc + b_ref[...], 0.0) * ns_ref[...]


_BN = 1000


def _dense(agg, nd, W, b, ns=None):
    col = pl.BlockSpec((_BN, 1), lambda i: (i, 0))
    specs = [
        col,
        pl.BlockSpec((_BN, D), lambda i: (i, 0)),
        pl.BlockSpec((D, D), lambda i: (0, 0)),
        pl.BlockSpec((1, D), lambda i: (0, 0)),
    ]
    args = [nd, agg, W, b.reshape(1, D)]
    body = _dense_kernel
    if ns is not None:
        specs.append(col)
        args.append(ns)
        body = _dense_scaled_kernel
    return pl.pallas_call(
        body,
        grid=(N // _BN,),
        in_specs=specs,
        out_specs=pl.BlockSpec((_BN, D), lambda i: (i, 0)),
        out_shape=jax.ShapeDtypeStruct((N, D), jnp.float32),
    )(*args)


def _rowscale_kernel(x_ref, ns_ref, o_ref):
    o_ref[...] = x_ref[...] * ns_ref[...]


def _rowscale(x, ns):
    return pl.pallas_call(
        _rowscale_kernel,
        grid=(N // _BN,),
        in_specs=[pl.BlockSpec((_BN, D), lambda i: (i, 0)),
                  pl.BlockSpec((_BN, 1), lambda i: (i, 0))],
        out_specs=pl.BlockSpec((_BN, D), lambda i: (i, 0)),
        out_shape=jax.ShapeDtypeStruct((N, D), jnp.float32),
    )(x, ns)


def kernel(x, edge_index, edge_weight, W1, b1, W2, b2):
    src = edge_index[0]
    dst = edge_index[1]
    ns, nd = _degree_norms(src, dst)
    ns2 = ns[:N].reshape(N, 1)
    nd2 = nd[:N].reshape(N, 1)
    xs = _rowscale(x, ns2)
    agg1 = _agg(xs, src, dst, edge_weight)[:N]
    h1s = _dense(agg1, nd2, W1, b1, ns=ns2)
    agg2 = _agg(h1s, src, dst, edge_weight)[:N]
    return _dense(agg2, nd2, W2, b2)
